# baseline (speedup 1.0000x reference)
import functools

import jax
import jax.numpy as jnp
from jax import lax
from jax.experimental import pallas as pl
from jax.experimental.pallas import tpu as pltpu
from jax.experimental.pallas import tpu_sc as plsc

VOCAB = 1000000
D = 32
DH = D // 2
B = 16384
NC = 2
NS = 16
NW = NC * NS
BPW = B // NW
L = 16


def _rsqrt16(x):
    i = lax.bitcast_convert_type(x, jnp.int32)
    i = jnp.int32(0x5F3759DF) - lax.shift_right_logical(i, 1)
    y = lax.bitcast_convert_type(i, jnp.float32)
    half = x * jnp.float32(0.5)
    for _ in range(3):
        y = y * (jnp.float32(1.5) - half * y * y)
    return y


def _worker_id():
    return lax.axis_index("s") * NC + lax.axis_index("c")


def _build():
    mesh = plsc.VectorSubcoreMesh(
        core_axis_name="c", subcore_axis_name="s", num_cores=NC, num_subcores=NS
    )

    @functools.partial(
        pl.kernel,
        out_type=(
            jax.ShapeDtypeStruct((B, DH), jnp.float32),
            jax.ShapeDtypeStruct((NW, L), jnp.float32),
        ),
        mesh=mesh,
        scratch_types=[
            pltpu.VMEM((BPW,), jnp.int32),
            pltpu.VMEM((BPW, D), jnp.float32),
            pltpu.VMEM((BPW, DH), jnp.float32),
            pltpu.VMEM((L,), jnp.float32),
            pltpu.SemaphoreType.DMA,
        ],
        compiler_params=pltpu.CompilerParams(use_tc_tiling_on_sc=False, needs_layout_passes=False),
    )
    def gather_part(ids_hbm, table_hbm, realun_hbm, part_hbm,
                    idx_v, rows_v, real_v, acc_v, sem):
        wid = _worker_id()
        base = wid * BPW
        pltpu.sync_copy(ids_hbm.at[pl.ds(base, BPW)], idx_v)
        pltpu.async_copy(table_hbm.at[idx_v], rows_v, sem).wait()

        col = lax.iota(jnp.int32, L) * 2

        def body(i, acc):
            row = jnp.full((L,), i, jnp.int32)
            g = plsc.load_gather(rows_v, [row, col])
            real_v[i, :] = g
            return acc + g * g

        acc = lax.fori_loop(0, BPW, body, jnp.zeros((L,), jnp.float32))
        acc_v[...] = acc
        pltpu.sync_copy(acc_v, part_hbm.at[wid])
        pltpu.sync_copy(real_v, realun_hbm.at[pl.ds(base, BPW)])

    @functools.partial(
        pl.kernel,
        out_type=jax.ShapeDtypeStruct((2, B, DH), jnp.float32),
        mesh=mesh,
        scratch_types=[
            pltpu.VMEM((NW, L), jnp.float32),
            pltpu.VMEM((BPW, DH), jnp.float32),
            pltpu.VMEM((BPW, DH), jnp.float32),
        ],
        compiler_params=pltpu.CompilerParams(use_tc_tiling_on_sc=False, needs_layout_passes=False),
    )
    def normalize(realun_hbm, part_hbm, out_hbm, part_v, real_v, zero_v):
        wid = _worker_id()
        base = wid * BPW
        pltpu.sync_copy(part_hbm, part_v)
        pltpu.sync_copy(realun_hbm.at[pl.ds(base, BPW)], real_v)

        def psum(j, t):
            return t + part_v[j, :]

        tot = lax.fori_loop(0, NW, psum, jnp.zeros((L,), jnp.float32))
        s = jnp.sum(tot) + jnp.float32(1e-12)
        r = _rsqrt16(lax.broadcast_in_dim(s, (L,), ()))
        z = jnp.zeros((L,), jnp.float32)

        def scale(i, _):
            real_v[i, :] = real_v[i, :] * r
            zero_v[i, :] = z
            return 0

        lax.fori_loop(0, BPW, scale, 0)
        pltpu.sync_copy(real_v, out_hbm.at[0].at[pl.ds(base, BPW)])
        pltpu.sync_copy(zero_v, out_hbm.at[1].at[pl.ds(base, BPW)])

    return gather_part, normalize


_KERNELS = None


def kernel(input_ids, table):
    global _KERNELS
    if _KERNELS is None:
        _KERNELS = _build()
    gather_part, normalize = _KERNELS
    realun, part = gather_part(input_ids, table)
    return normalize(realun, part)


# speedup vs baseline: 4.2286x; 4.2286x over previous
import functools

import jax
import jax.numpy as jnp
from jax import lax
from jax.experimental import pallas as pl
from jax.experimental.pallas import tpu as pltpu
from jax.experimental.pallas import tpu_sc as plsc

VOCAB = 1000000
D = 32
DH = D // 2
B = 16384
NC = 2
NS = 16
NW = NC * NS
BPW = B // NW
L = 16
K = 4
G = BPW // K


def _rsqrt16(x):
    i = lax.bitcast_convert_type(x, jnp.int32)
    i = jnp.int32(0x5F3759DF) - lax.shift_right_logical(i, 1)
    y = lax.bitcast_convert_type(i, jnp.float32)
    half = x * jnp.float32(0.5)
    for _ in range(3):
        y = y * (jnp.float32(1.5) - half * y * y)
    return y


def _worker_id():
    return lax.axis_index("s") * NC + lax.axis_index("c")


def _build():
    mesh = plsc.VectorSubcoreMesh(
        core_axis_name="c", subcore_axis_name="s", num_cores=NC, num_subcores=NS
    )
    params = pltpu.CompilerParams(needs_layout_passes=False)

    @functools.partial(
        pl.kernel,
        out_type=(
            jax.ShapeDtypeStruct((B, DH), jnp.float32),
            jax.ShapeDtypeStruct((NW, L), jnp.float32),
        ),
        mesh=mesh,
        scratch_types=[
            pltpu.VMEM((BPW,), jnp.int32),
            pltpu.VMEM((2, K, L, 128), jnp.float32),
            pltpu.VMEM((BPW, DH), jnp.float32),
            pltpu.VMEM((L,), jnp.float32),
            pltpu.SemaphoreType.DMA,
            pltpu.SemaphoreType.DMA,
        ],
        compiler_params=params,
    )
    def gather_part(ids_hbm, tablet_hbm, realun_hbm, part_hbm,
                    ids_v, blk_v, cols_v, acc_v, sem0, sem1):
        wid = _worker_id()
        base = wid * BPW
        pltpu.sync_copy(ids_hbm.at[pl.ds(base, BPW)], ids_v)

        dvec = lax.iota(jnp.int32, L) * 2
        iot = lax.iota(jnp.int32, L)
        sems = (sem0, sem1)

        def src(idv):
            q128 = pl.multiple_of(
                lax.shift_left(lax.shift_right_logical(idv, 7), 7), 128
            )
            return tablet_hbm.at[:, pl.ds(q128, 128)].at[dvec]

        def fire(ids16, lane0, par):
            for j in range(K):
                pltpu.async_copy(
                    src(ids16[lane0 + j]), blk_v.at[par, j], sems[par]
                )

        def drain_use(ids16, lane0, c, par, acc):
            for j in range(K):
                i = c * K + j
                idv = ids16[lane0 + j]
                pltpu.make_async_copy(
                    src(idv), blk_v.at[par, j], sems[par]
                ).wait()
                r = jnp.full((L,), idv & 127, jnp.int32)
                gth = plsc.load_gather(
                    blk_v,
                    [jnp.full((L,), par, jnp.int32),
                     jnp.full((L,), j, jnp.int32), iot, r],
                )
                cols_v[i, :] = gth
                acc = acc + gth * gth
            return acc

        NQ = G // 4
        ids16_0 = ids_v[pl.ds(0, 16)]
        fire(ids16_0, 0, 0)
        fire(ids16_0, K, 1)

        def quad(q, acc):
            c0 = 4 * q
            ids16 = ids_v[pl.ds(16 * q, 16)]
            acc = drain_use(ids16, 0, c0, 0, acc)
            fire(ids16, 2 * K, 0)
            acc = drain_use(ids16, K, c0 + 1, 1, acc)
            fire(ids16, 3 * K, 1)
            acc = drain_use(ids16, 2 * K, c0 + 2, 0, acc)

            @pl.when(q + 1 < NQ)
            def _():
                ids16n = ids_v[pl.ds(16 * (q + 1), 16)]
                fire(ids16n, 0, 0)

            acc = drain_use(ids16, 3 * K, c0 + 3, 1, acc)

            @pl.when(q + 1 < NQ)
            def _():
                ids16n = ids_v[pl.ds(16 * (q + 1), 16)]
                fire(ids16n, K, 1)

            return acc

        acc = lax.fori_loop(0, NQ, quad, jnp.zeros((L,), jnp.float32))
        acc_v[...] = acc
        pltpu.sync_copy(acc_v, part_hbm.at[wid])
        pltpu.sync_copy(cols_v, realun_hbm.at[pl.ds(base, BPW)])

    @functools.partial(
        pl.kernel,
        out_type=jax.ShapeDtypeStruct((2, B, DH), jnp.float32),
        mesh=mesh,
        scratch_types=[
            pltpu.VMEM((NW, L), jnp.float32),
            pltpu.VMEM((BPW, DH), jnp.float32),
        ],
        compiler_params=params,
    )
    def normalize(realun_hbm, part_hbm, out_hbm, part_v, real_v):
        wid = _worker_id()
        base = wid * BPW
        pltpu.sync_copy(part_hbm, part_v)
        pltpu.sync_copy(realun_hbm.at[pl.ds(base, BPW)], real_v)

        def psum(j, t):
            return t + part_v[j, :]

        tot = lax.fori_loop(0, NW, psum, jnp.zeros((L,), jnp.float32))
        s = jnp.sum(tot) + jnp.float32(1e-12)
        r = _rsqrt16(lax.broadcast_in_dim(s, (L,), ()))
        z = jnp.zeros((L,), jnp.float32)

        def scale(i, _):
            real_v[i, :] = real_v[i, :] * r
            return 0

        lax.fori_loop(0, BPW, scale, 0)
        pltpu.sync_copy(real_v, out_hbm.at[0].at[pl.ds(base, BPW)])

        def zero(i, _):
            real_v[i, :] = z
            return 0

        lax.fori_loop(0, BPW, zero, 0)
        pltpu.sync_copy(real_v, out_hbm.at[1].at[pl.ds(base, BPW)])

    return gather_part, normalize


_KERNELS = None


def kernel(input_ids, table):
    global _KERNELS
    if _KERNELS is None:
        _KERNELS = _build()
    gather_part, normalize = _KERNELS
    realun, part = gather_part(input_ids, table.T)
    return normalize(realun, part)


# speedup vs baseline: 4.5900x; 1.0855x over previous
import functools

import jax
import jax.numpy as jnp
from jax import lax
from jax.experimental import pallas as pl
from jax.experimental.pallas import tpu as pltpu
from jax.experimental.pallas import tpu_sc as plsc

VOCAB = 1000000
D = 32
DH = D // 2
B = 16384
NC = 2
NS = 16
NW = NC * NS
BPW = B // NW
L = 16
K = 4
G = BPW // K


def _rsqrt16(x):
    i = lax.bitcast_convert_type(x, jnp.int32)
    i = jnp.int32(0x5F3759DF) - lax.shift_right_logical(i, 1)
    y = lax.bitcast_convert_type(i, jnp.float32)
    half = x * jnp.float32(0.5)
    for _ in range(3):
        y = y * (jnp.float32(1.5) - half * y * y)
    return y


def _build():
    mesh = plsc.VectorSubcoreMesh(
        core_axis_name="c", subcore_axis_name="s", num_cores=NC, num_subcores=NS
    )
    params = pltpu.CompilerParams(needs_layout_passes=False)

    @functools.partial(
        pl.kernel,
        out_type=(
            jax.ShapeDtypeStruct((2, B, DH), jnp.float32),
            jax.ShapeDtypeStruct((NW, L), jnp.float32),
        ),
        mesh=mesh,
        scratch_types=[
            pltpu.VMEM((BPW,), jnp.int32),
            pltpu.VMEM((2, K, L, 128), jnp.float32),
            pltpu.VMEM((BPW, DH), jnp.float32),
            pltpu.VMEM((L,), jnp.float32),
            pltpu.VMEM((NW, L), jnp.float32),
            pltpu.SemaphoreType.DMA,
            pltpu.SemaphoreType.DMA,
            pltpu.SemaphoreType.REGULAR,
        ],
        compiler_params=params,
    )
    def fused(ids_hbm, tablet_hbm, out_hbm, part_hbm,
              ids_v, blk_v, cols_v, acc_v, part_v, sem0, sem1, xsem):
        cid = lax.axis_index("c")
        sid = lax.axis_index("s")
        wid = sid * NC + cid
        base = wid * BPW
        pltpu.sync_copy(ids_hbm.at[pl.ds(base, BPW)], ids_v)

        dvec = lax.iota(jnp.int32, L) * 2
        iot = lax.iota(jnp.int32, L)
        sems = (sem0, sem1)

        def src(idv):
            q128 = pl.multiple_of(
                lax.shift_left(lax.shift_right_logical(idv, 7), 7), 128
            )
            return tablet_hbm.at[:, pl.ds(q128, 128)].at[dvec]

        def fire(ids16, lane0, par):
            for j in range(K):
                pltpu.async_copy(
                    src(ids16[lane0 + j]), blk_v.at[par, j], sems[par]
                )

        def drain_use(ids16, lane0, c, par, acc):
            for j in range(K):
                i = c * K + j
                idv = ids16[lane0 + j]
                pltpu.make_async_copy(
                    src(idv), blk_v.at[par, j], sems[par]
                ).wait()
                r = jnp.full((L,), idv & 127, jnp.int32)
                gth = plsc.load_gather(
                    blk_v,
                    [jnp.full((L,), par, jnp.int32),
                     jnp.full((L,), j, jnp.int32), iot, r],
                )
                cols_v[i, :] = gth
                acc = acc + gth * gth
            return acc

        NQ = G // 4
        ids16_0 = ids_v[pl.ds(0, 16)]
        fire(ids16_0, 0, 0)
        fire(ids16_0, K, 1)

        def quad(q, acc):
            c0 = 4 * q
            ids16 = ids_v[pl.ds(16 * q, 16)]
            acc = drain_use(ids16, 0, c0, 0, acc)
            fire(ids16, 2 * K, 0)
            acc = drain_use(ids16, K, c0 + 1, 1, acc)
            fire(ids16, 3 * K, 1)
            acc = drain_use(ids16, 2 * K, c0 + 2, 0, acc)

            @pl.when(q + 1 < NQ)
            def _():
                ids16n = ids_v[pl.ds(16 * (q + 1), 16)]
                fire(ids16n, 0, 0)

            acc = drain_use(ids16, 3 * K, c0 + 3, 1, acc)

            @pl.when(q + 1 < NQ)
            def _():
                ids16n = ids_v[pl.ds(16 * (q + 1), 16)]
                fire(ids16n, K, 1)

            return acc

        acc = lax.fori_loop(0, NQ, quad, jnp.zeros((L,), jnp.float32))
        acc_v[...] = acc
        pltpu.sync_copy(acc_v, part_hbm.at[wid])

        plsc.subcore_barrier()
        pltpu.semaphore_signal(xsem, 1, core_index=1 - cid)
        pl.semaphore_wait(xsem, 1)

        pltpu.sync_copy(part_hbm, part_v)

        def psum(j, t):
            return t + part_v[j, :]

        tot = lax.fori_loop(0, NW, psum, jnp.zeros((L,), jnp.float32))
        s = jnp.sum(tot) + jnp.float32(1e-12)
        r = _rsqrt16(lax.broadcast_in_dim(s, (L,), ()))
        z = jnp.zeros((L,), jnp.float32)

        def scale(i, _):
            cols_v[i, :] = cols_v[i, :] * r
            return 0

        lax.fori_loop(0, BPW, scale, 0)
        pltpu.sync_copy(cols_v, out_hbm.at[0].at[pl.ds(base, BPW)])

        def zero(i, _):
            cols_v[i, :] = z
            return 0

        lax.fori_loop(0, BPW, zero, 0)
        pltpu.sync_copy(cols_v, out_hbm.at[1].at[pl.ds(base, BPW)])

    return fused


_FUSED = None


def kernel(input_ids, table):
    global _FUSED
    if _FUSED is None:
        _FUSED = _build()
    out, _ = _FUSED(input_ids, table.T)
    return out


# speedup vs baseline: 5.5163x; 1.2018x over previous
import functools

import jax
import jax.numpy as jnp
from jax import lax
from jax.experimental import pallas as pl
from jax.experimental.pallas import tpu as pltpu
from jax.experimental.pallas import tpu_sc as plsc

VOCAB = 1000000
D = 32
DH = D // 2
B = 16384
NC = 2
NS = 16
NW = NC * NS
BPW = B // NW
L = 16
K = 4
G = BPW // K


def _rsqrt16(x):
    i = lax.bitcast_convert_type(x, jnp.int32)
    i = jnp.int32(0x5F3759DF) - lax.shift_right_logical(i, 1)
    y = lax.bitcast_convert_type(i, jnp.float32)
    half = x * jnp.float32(0.5)
    for _ in range(3):
        y = y * (jnp.float32(1.5) - half * y * y)
    return y


def _build():
    mesh = plsc.VectorSubcoreMesh(
        core_axis_name="c", subcore_axis_name="s", num_cores=NC, num_subcores=NS
    )
    params = pltpu.CompilerParams(needs_layout_passes=False)

    @functools.partial(
        pl.kernel,
        out_type=(
            jax.ShapeDtypeStruct((2, B, DH), jnp.float32),
            jax.ShapeDtypeStruct((NW, L), jnp.float32),
        ),
        mesh=mesh,
        scratch_types=[
            pltpu.VMEM((BPW,), jnp.int32),
            pltpu.VMEM((4, K, L, 128), jnp.float32),
            pltpu.VMEM((BPW, DH), jnp.float32),
            pltpu.VMEM((L,), jnp.float32),
            pltpu.VMEM((NW, L), jnp.float32),
            pltpu.SemaphoreType.DMA,
            pltpu.SemaphoreType.DMA,
            pltpu.SemaphoreType.DMA,
            pltpu.SemaphoreType.DMA,
            pltpu.SemaphoreType.REGULAR,
        ],
        compiler_params=params,
    )
    def fused(ids_hbm, tablet_hbm, out_hbm, part_hbm,
              ids_v, blk_v, cols_v, acc_v, part_v, sem0, sem1, sem2, sem3,
              xsem):
        cid = lax.axis_index("c")
        sid = lax.axis_index("s")
        wid = sid * NC + cid
        base = wid * BPW
        pltpu.sync_copy(ids_hbm.at[pl.ds(base, BPW)], ids_v)

        dvec = lax.iota(jnp.int32, L) * 2
        iot = lax.iota(jnp.int32, L)
        sems = (sem0, sem1, sem2, sem3)

        def src(idv):
            q128 = pl.multiple_of(
                lax.shift_left(lax.shift_right_logical(idv, 7), 7), 128
            )
            return tablet_hbm.at[:, pl.ds(q128, 128)].at[dvec]

        def fire(ids16, lane0, par):
            for j in range(K):
                pltpu.async_copy(
                    src(ids16[lane0 + j]), blk_v.at[par, j], sems[par]
                )

        def drain_use(ids16, lane0, c, par, acc):
            for j in range(K):
                i = c * K + j
                idv = ids16[lane0 + j]
                pltpu.make_async_copy(
                    src(idv), blk_v.at[par, j], sems[par]
                ).wait()
                r = jnp.full((L,), idv & 127, jnp.int32)
                gth = plsc.load_gather(
                    blk_v,
                    [jnp.full((L,), par, jnp.int32),
                     jnp.full((L,), j, jnp.int32), iot, r],
                )
                cols_v[i, :] = gth
                acc = acc + gth * gth
            return acc

        NQ = G // 4
        ids16_0 = ids_v[pl.ds(0, 16)]
        fire(ids16_0, 0, 0)
        fire(ids16_0, K, 1)
        fire(ids16_0, 2 * K, 2)

        def quad(q, acc):
            c0 = 4 * q
            ids16 = ids_v[pl.ds(16 * q, 16)]
            acc = drain_use(ids16, 0, c0, 0, acc)
            fire(ids16, 3 * K, 3)
            acc = drain_use(ids16, K, c0 + 1, 1, acc)

            @pl.when(q + 1 < NQ)
            def _():
                ids16n = ids_v[pl.ds(16 * (q + 1), 16)]
                fire(ids16n, 0, 0)

            acc = drain_use(ids16, 2 * K, c0 + 2, 2, acc)

            @pl.when(q + 1 < NQ)
            def _():
                ids16n = ids_v[pl.ds(16 * (q + 1), 16)]
                fire(ids16n, K, 1)

            acc = drain_use(ids16, 3 * K, c0 + 3, 3, acc)

            @pl.when(q + 1 < NQ)
            def _():
                ids16n = ids_v[pl.ds(16 * (q + 1), 16)]
                fire(ids16n, 2 * K, 2)

            return acc

        acc = lax.fori_loop(0, NQ, quad, jnp.zeros((L,), jnp.float32))
        acc_v[...] = acc
        pltpu.sync_copy(acc_v, part_hbm.at[wid])

        plsc.subcore_barrier()
        pltpu.semaphore_signal(xsem, 1, core_index=1 - cid)
        pl.semaphore_wait(xsem, 1)

        pltpu.sync_copy(part_hbm, part_v)

        def psum(j, t):
            return t + part_v[j, :]

        tot = lax.fori_loop(0, NW, psum, jnp.zeros((L,), jnp.float32))
        s = jnp.sum(tot) + jnp.float32(1e-12)
        r = _rsqrt16(lax.broadcast_in_dim(s, (L,), ()))
        z = jnp.zeros((L,), jnp.float32)

        def scale(i, _):
            for u in range(4):
                cols_v[4 * i + u, :] = cols_v[4 * i + u, :] * r
            return 0

        lax.fori_loop(0, BPW // 4, scale, 0)
        pltpu.sync_copy(cols_v, out_hbm.at[0].at[pl.ds(base, BPW)])

        def zero(i, _):
            for u in range(4):
                cols_v[4 * i + u, :] = z
            return 0

        lax.fori_loop(0, BPW // 4, zero, 0)
        pltpu.sync_copy(cols_v, out_hbm.at[1].at[pl.ds(base, BPW)])

    return fused


_FUSED = None


def kernel(input_ids, table):
    global _FUSED
    if _FUSED is None:
        _FUSED = _build()
    out, _ = _FUSED(input_ids, table.T)
    return out


# speedup vs baseline: 5.5339x; 1.0032x over previous
import functools

import jax
import jax.numpy as jnp
from jax import lax
from jax.experimental import pallas as pl
from jax.experimental.pallas import tpu as pltpu
from jax.experimental.pallas import tpu_sc as plsc

VOCAB = 1000000
D = 32
DH = D // 2
B = 16384
NC = 2
NS = 16
NW = NC * NS
BPW = B // NW
L = 16
K = 4
G = BPW // K


def _rsqrt16(x):
    i = lax.bitcast_convert_type(x, jnp.int32)
    i = jnp.int32(0x5F3759DF) - lax.shift_right_logical(i, 1)
    y = lax.bitcast_convert_type(i, jnp.float32)
    half = x * jnp.float32(0.5)
    for _ in range(3):
        y = y * (jnp.float32(1.5) - half * y * y)
    return y


def _build():
    mesh = plsc.VectorSubcoreMesh(
        core_axis_name="c", subcore_axis_name="s", num_cores=NC, num_subcores=NS
    )
    params = pltpu.CompilerParams(needs_layout_passes=False)

    @functools.partial(
        pl.kernel,
        out_type=(
            jax.ShapeDtypeStruct((2, B, DH), jnp.float32),
            jax.ShapeDtypeStruct((NW, L), jnp.float32),
        ),
        mesh=mesh,
        scratch_types=[
            pltpu.VMEM((BPW,), jnp.int32),
            pltpu.VMEM((L,), jnp.int32),
            pltpu.VMEM((4, K, L, 128), jnp.float32),
            pltpu.VMEM((BPW, DH), jnp.float32),
            pltpu.VMEM((L,), jnp.float32),
            pltpu.VMEM((NW, L), jnp.float32),
            pltpu.SemaphoreType.DMA,
            pltpu.SemaphoreType.DMA,
            pltpu.SemaphoreType.DMA,
            pltpu.SemaphoreType.DMA,
            pltpu.SemaphoreType.REGULAR,
        ],
        compiler_params=params,
    )
    def fused(ids_hbm, tablet_hbm, out_hbm, part_hbm,
              ids_v, dvec_v, blk_v, cols_v, acc_v, part_v,
              sem0, sem1, sem2, sem3, xsem):
        cid = lax.axis_index("c")
        sid = lax.axis_index("s")
        wid = sid * NC + cid
        base = wid * BPW
        pltpu.sync_copy(ids_hbm.at[pl.ds(base, BPW)], ids_v)

        iot = lax.iota(jnp.int32, L)
        dvec_v[...] = iot * 2
        sems = (sem0, sem1, sem2, sem3)

        def src(idv):
            q128 = pl.multiple_of(
                lax.shift_left(lax.shift_right_logical(idv, 7), 7), 128
            )
            return tablet_hbm.at[:, pl.ds(q128, 128)].at[dvec_v]

        def fire(ids16, lane0, par):
            for j in range(K):
                pltpu.async_copy(
                    src(ids16[lane0 + j]), blk_v.at[par, j], sems[par]
                )

        def drain_use(ids16, lane0, c, par, acc):
            for j in range(K):
                i = c * K + j
                idv = ids16[lane0 + j]
                pltpu.make_async_copy(
                    src(idv), blk_v.at[par, j], sems[par]
                ).wait()
                r = jnp.full((L,), idv & 127, jnp.int32)
                gth = plsc.load_gather(
                    blk_v,
                    [jnp.full((L,), par, jnp.int32),
                     jnp.full((L,), j, jnp.int32), iot, r],
                )
                cols_v[i, :] = gth
                acc = acc + gth * gth
            return acc

        NQ = G // 4
        ids16_0 = ids_v[pl.ds(0, 16)]
        fire(ids16_0, 0, 0)
        fire(ids16_0, K, 1)
        fire(ids16_0, 2 * K, 2)

        def quad(q, acc):
            c0 = 4 * q
            ids16 = ids_v[pl.ds(16 * q, 16)]
            acc = drain_use(ids16, 0, c0, 0, acc)
            fire(ids16, 3 * K, 3)
            acc = drain_use(ids16, K, c0 + 1, 1, acc)

            @pl.when(q + 1 < NQ)
            def _():
                ids16n = ids_v[pl.ds(16 * (q + 1), 16)]
                fire(ids16n, 0, 0)

            acc = drain_use(ids16, 2 * K, c0 + 2, 2, acc)

            @pl.when(q + 1 < NQ)
            def _():
                ids16n = ids_v[pl.ds(16 * (q + 1), 16)]
                fire(ids16n, K, 1)

            acc = drain_use(ids16, 3 * K, c0 + 3, 3, acc)

            @pl.when(q + 1 < NQ)
            def _():
                ids16n = ids_v[pl.ds(16 * (q + 1), 16)]
                fire(ids16n, 2 * K, 2)

            return acc

        acc = lax.fori_loop(0, NQ, quad, jnp.zeros((L,), jnp.float32))
        acc_v[...] = acc
        pltpu.sync_copy(acc_v, part_hbm.at[wid])

        plsc.subcore_barrier()
        pltpu.semaphore_signal(xsem, 1, core_index=1 - cid)
        pl.semaphore_wait(xsem, 1)

        pltpu.sync_copy(part_hbm, part_v)

        def psum(j, t):
            return t + part_v[j, :]

        tot = lax.fori_loop(0, NW, psum, jnp.zeros((L,), jnp.float32))
        s = jnp.sum(tot) + jnp.float32(1e-12)
        r = _rsqrt16(lax.broadcast_in_dim(s, (L,), ()))
        z = jnp.zeros((L,), jnp.float32)

        def scale(i, _):
            for u in range(4):
                cols_v[4 * i + u, :] = cols_v[4 * i + u, :] * r
            return 0

        lax.fori_loop(0, BPW // 4, scale, 0)
        pltpu.sync_copy(cols_v, out_hbm.at[0].at[pl.ds(base, BPW)])

        def zero(i, _):
            for u in range(4):
                cols_v[4 * i + u, :] = z
            return 0

        lax.fori_loop(0, BPW // 4, zero, 0)
        pltpu.sync_copy(cols_v, out_hbm.at[1].at[pl.ds(base, BPW)])

    return fused


_FUSED = None


def kernel(input_ids, table):
    global _FUSED
    if _FUSED is None:
        _FUSED = _build()
    out, _ = _FUSED(input_ids, table.T)
    return out
